# baseline (speedup 1.0000x reference)
import jax
import jax.numpy as jnp
from jax import lax
from jax.experimental import pallas as pl
from jax.experimental.pallas import tpu as pltpu
from jax.experimental.pallas import tpu_sc as plsc

_NUM_CORES = 2
_NUM_SUBCORES = 16
_NUM_WORKERS = _NUM_CORES * _NUM_SUBCORES


def _make_gather(batch: int, half_vocab: int, dim: int):
    assert batch % (8 * _NUM_WORKERS) == 0
    b_per_w = batch // _NUM_WORKERS
    chunk = b_per_w // 2

    mesh = plsc.VectorSubcoreMesh(core_axis_name="c", subcore_axis_name="s")

    def body(ta_hbm, tb_hbm, idx_hbm, out_hbm,
             idx_v, ia_v, ib_v, ra_v, rb_v, out_v, sem_a, sem_b):
        wid = lax.axis_index("s") * _NUM_CORES + lax.axis_index("c")
        base = wid * b_per_w
        pltpu.sync_copy(idx_hbm.at[pl.ds(base, b_per_w)], idx_v)

        @pl.loop(0, b_per_w, step=chunk)
        def _(c):
            @pl.loop(0, chunk, step=16)
            def _(t):
                v = idx_v[pl.ds(c + t, 16)]
                in_b = v >= half_vocab
                ia_v[pl.ds(t, 16)] = jnp.where(in_b, 0, v)
                ib_v[pl.ds(t, 16)] = jnp.where(in_b, v - half_vocab, 0)

            ga = pltpu.make_async_copy(ta_hbm.at[ia_v], ra_v, sem_a)
            gb = pltpu.make_async_copy(tb_hbm.at[ib_v], rb_v, sem_b)
            ga.start()
            gb.start()
            ga.wait()
            gb.wait()

            @pl.loop(0, chunk, step=16)
            def _(t):
                v = idx_v[pl.ds(c + t, 16)]
                for k in range(16):
                    sel = v[k] >= half_vocab
                    for h in range(0, dim, 16):
                        a = ra_v[t + k, pl.ds(h, 16)]
                        b = rb_v[t + k, pl.ds(h, 16)]
                        out_v[t + k, pl.ds(h, 16)] = jnp.where(sel, b, a)

            pltpu.sync_copy(out_v, out_hbm.at[pl.ds(base + c, chunk)])

    return pl.kernel(
        body,
        mesh=mesh,
        out_type=jax.ShapeDtypeStruct((batch, dim), jnp.float32),
        scratch_types=[
            pltpu.VMEM((b_per_w,), jnp.int32),
            pltpu.VMEM((chunk,), jnp.int32),
            pltpu.VMEM((chunk,), jnp.int32),
            pltpu.VMEM((chunk, dim), jnp.float32),
            pltpu.VMEM((chunk, dim), jnp.float32),
            pltpu.VMEM((chunk, dim), jnp.float32),
            pltpu.SemaphoreType.DMA,
            pltpu.SemaphoreType.DMA,
        ],
        compiler_params=pltpu.CompilerParams(use_tc_tiling_on_sc=False),
    )


@jax.jit
def kernel(indices, table):
    batch = indices.shape[0]
    vocab, dim = table.shape
    half_vocab = vocab // 2
    idx = indices.astype(jnp.int32)
    table_a = lax.slice_in_dim(table, 0, half_vocab, axis=0)
    table_b = lax.slice_in_dim(table, half_vocab, vocab, axis=0)
    return _make_gather(batch, half_vocab, dim)(table_a, table_b, idx)


# speedup vs baseline: 2.9401x; 2.9401x over previous
import jax
import jax.numpy as jnp
from jax import lax
from jax.experimental import pallas as pl
from jax.experimental.pallas import tpu as pltpu
from jax.experimental.pallas import tpu_sc as plsc

_NUM_CORES = 2
_NUM_SUBCORES = 16
_NUM_WORKERS = _NUM_CORES * _NUM_SUBCORES


def _make_gather(batch: int, dim: int):
    assert batch % (8 * _NUM_WORKERS) == 0
    b_per_w = batch // _NUM_WORKERS

    mesh = plsc.VectorSubcoreMesh(core_axis_name="c", subcore_axis_name="s")

    def body(table_hbm, idx_hbm, out_hbm, idx_v, rows_v, sem):
        wid = lax.axis_index("s") * _NUM_CORES + lax.axis_index("c")
        base = wid * b_per_w
        out_slice = out_hbm.at[pl.ds(base, b_per_w)]
        pltpu.sync_copy(idx_hbm.at[pl.ds(base, b_per_w)], idx_v)

        @pl.loop(0, b_per_w, step=16)
        def _(j):
            v = idx_v[pl.ds(j, 16)]
            for k in range(16):
                pltpu.make_async_copy(
                    table_hbm.at[v[k]], rows_v.at[j + k], sem
                ).start()

        pltpu.make_async_copy(out_slice, rows_v, sem).wait()
        pltpu.sync_copy(rows_v, out_slice)

    return pl.kernel(
        body,
        mesh=mesh,
        out_type=jax.ShapeDtypeStruct((batch, dim), jnp.float32),
        scratch_types=[
            pltpu.VMEM((b_per_w,), jnp.int32),
            pltpu.VMEM((b_per_w, dim), jnp.float32),
            pltpu.SemaphoreType.DMA,
        ],
    )


@jax.jit
def kernel(indices, table):
    batch = indices.shape[0]
    dim = table.shape[1]
    idx = indices.astype(jnp.int32)
    return _make_gather(batch, dim)(table, idx)
